# baseline (speedup 1.0000x reference)
import jax
import jax.numpy as jnp
from jax.experimental import pallas as pl
from jax.experimental.pallas import tpu as pltpu

_BM = 400
_NBUF = 3


def _gcn_body(x_ref, w_ref, adj_hbm, out_hbm, h_ref, out_vmem, bufs, in_sems,
              out_sems):
    n, _ = x_ref.shape
    bm = bufs.shape[1]
    nblocks = n // bm

    def in_copy(b, slot):
        return pltpu.make_async_copy(
            adj_hbm.at[pl.ds(b * bm, bm), :], bufs.at[slot], in_sems.at[slot]
        )

    def out_copy(b, slot):
        return pltpu.make_async_copy(
            out_vmem.at[slot], out_hbm.at[pl.ds(b * bm, bm), :],
            out_sems.at[slot],
        )

    for b in range(_NBUF - 1):
        in_copy(b, b).start()

    h_ref[...] = jax.lax.dot_general(
        x_ref[...], w_ref[...],
        dimension_numbers=(((1,), (1,)), ((), ())),
        preferred_element_type=jnp.float32,
    )

    def step(b, _):
        slot = jax.lax.rem(b, _NBUF)
        in_copy(b, slot).wait()

        @pl.when(b >= 2)
        def _():
            out_copy(b - 2, jax.lax.rem(b, 2)).wait()

        out_vmem[jax.lax.rem(b, 2)] = jnp.dot(
            bufs[slot], h_ref[...], preferred_element_type=jnp.float32
        )
        out_copy(b, jax.lax.rem(b, 2)).start()

        @pl.when(b + _NBUF - 1 < nblocks)
        def _():
            in_copy(b + _NBUF - 1, jax.lax.rem(b + _NBUF - 1, _NBUF)).start()

        return 0

    jax.lax.fori_loop(0, nblocks, step, 0)
    out_copy(nblocks - 2, jax.lax.rem(nblocks - 2, 2)).wait()
    out_copy(nblocks - 1, jax.lax.rem(nblocks - 1, 2)).wait()


def kernel(x, adj_norm, W):
    n, d_in = x.shape
    d_out = W.shape[0]
    bm = _BM if n % _BM == 0 else n
    return pl.pallas_call(
        _gcn_body,
        in_specs=[
            pl.BlockSpec(memory_space=pltpu.VMEM),
            pl.BlockSpec(memory_space=pltpu.VMEM),
            pl.BlockSpec(memory_space=pl.ANY),
        ],
        out_specs=pl.BlockSpec(memory_space=pl.ANY),
        out_shape=jax.ShapeDtypeStruct((n, d_out), jnp.float32),
        scratch_shapes=[
            pltpu.VMEM((n, d_out), jnp.float32),
            pltpu.VMEM((2, bm, d_out), jnp.float32),
            pltpu.VMEM((_NBUF, bm, n), jnp.float32),
            pltpu.SemaphoreType.DMA((_NBUF,)),
            pltpu.SemaphoreType.DMA((2,)),
        ],
        compiler_params=pltpu.CompilerParams(
            vmem_limit_bytes=64 * 1024 * 1024,
        ),
    )(x, W, adj_norm)


# speedup vs baseline: 1.0169x; 1.0169x over previous
import jax
import jax.numpy as jnp
from jax.experimental import pallas as pl
from jax.experimental.pallas import tpu as pltpu

_BM = 400


def _gcn_body(x_ref, w_ref, adj_ref, out_ref, h_ref):
    i = pl.program_id(0)

    @pl.when(i == 0)
    def _project():
        h_ref[...] = jax.lax.dot_general(
            x_ref[...], w_ref[...],
            dimension_numbers=(((1,), (1,)), ((), ())),
            preferred_element_type=jnp.float32,
        )

    out_ref[...] = jnp.dot(
        adj_ref[...], h_ref[...], preferred_element_type=jnp.float32
    )


def kernel(x, adj_norm, W):
    n, d_in = x.shape
    d_out = W.shape[0]
    bm = _BM if n % _BM == 0 else n
    grid = (n // bm,)
    return pl.pallas_call(
        _gcn_body,
        grid=grid,
        in_specs=[
            pl.BlockSpec((n, d_in), lambda i: (0, 0)),
            pl.BlockSpec((d_out, d_in), lambda i: (0, 0)),
            pl.BlockSpec((bm, n), lambda i: (i, 0)),
        ],
        out_specs=pl.BlockSpec((bm, d_out), lambda i: (i, 0)),
        out_shape=jax.ShapeDtypeStruct((n, d_out), jnp.float32),
        scratch_shapes=[pltpu.VMEM((n, d_out), jnp.float32)],
    )(x, W, adj_norm)


# speedup vs baseline: 1.0182x; 1.0013x over previous
import jax
import jax.numpy as jnp
from jax.experimental import pallas as pl

_BM = 400


def _gcn_body(x_ref, w_ref, adj_ref, out_ref):
    t = jnp.dot(adj_ref[...], x_ref[...], preferred_element_type=jnp.float32)
    out_ref[...] = jax.lax.dot_general(
        t, w_ref[...],
        dimension_numbers=(((1,), (1,)), ((), ())),
        preferred_element_type=jnp.float32,
    )


def kernel(x, adj_norm, W):
    n, d_in = x.shape
    d_out = W.shape[0]
    bm = _BM if n % _BM == 0 else n
    grid = (n // bm,)
    return pl.pallas_call(
        _gcn_body,
        grid=grid,
        in_specs=[
            pl.BlockSpec((n, d_in), lambda i: (0, 0)),
            pl.BlockSpec((d_out, d_in), lambda i: (0, 0)),
            pl.BlockSpec((bm, n), lambda i: (i, 0)),
        ],
        out_specs=pl.BlockSpec((bm, d_out), lambda i: (i, 0)),
        out_shape=jax.ShapeDtypeStruct((n, d_out), jnp.float32),
    )(x, W, adj_norm)
